# baseline (speedup 1.0000x reference)
import functools
import jax
import jax.numpy as jnp
from jax import lax
from jax.experimental import pallas as pl
from jax.experimental.pallas import tpu as pltpu
from jax.experimental.pallas import tpu_sc as plsc

P = 97
OP_ADD = P
OP_SUB = P + 1
OP_MUL = P + 2
PAD = P + 6
N = 32768
NT = 16
C = N // NT
L = 16
NV = C // L
HB2 = C + 48
FPAD = 24
TPAD = 40
OPAD = 2080

_i32 = jnp.int32
_f32 = jnp.float32


def _sc_body(tok_hbm, val_hbm, cu_hbm, out_tok, out_val, ncu_hbm,
             tbuf, vbuf, sfl, redf, ktok, kval, npb, obt, obv,
             padb, zfb, cuv, cuix, ncub, fragt, fragv, fragix, smem,
             sem_in, sem_fill, sem_sc):
    w = lax.axis_index("s")
    base = pl.multiple_of(w * C, C)
    ii = lax.iota(_i32, L)
    zz = jnp.zeros((L,), _i32)

    def _zsm(i, _):
        smem[i] = 0
        return 0
    lax.fori_loop(0, 18, _zsm, 0)

    oin = pl.multiple_of(jnp.clip(w * C - 16, 0, N - HB2), 8)
    dd = (w * C - 16) - oin
    h_t = pltpu.async_copy(tok_hbm.at[pl.ds(oin, HB2)], tbuf, sem_in)
    h_v = pltpu.async_copy(val_hbm.at[pl.ds(oin, HB2)], vbuf, sem_in)
    pltpu.sync_copy(cu_hbm, cuv)

    @plsc.parallel_loop(0, C, L)
    def _prep(k):
        padb[pl.ds(k, L)] = jnp.full((L,), PAD, _i32)
        zfb[pl.ds(k, L)] = jnp.zeros((L,), _f32)
        sfl[pl.ds(k, L)] = zz
    @plsc.parallel_loop(C, HB2, L)
    def _prep2(k):
        sfl[pl.ds(k, L)] = zz

    h_t.wait()
    h_v.wait()

    f_t = pltpu.async_copy(padb, out_tok.at[pl.ds(base, C)], sem_fill)
    f_v = pltpu.async_copy(zfb, out_val.at[pl.ds(base, C)], sem_fill)

    cuvec = cuv[...]
    kidx = cuvec - base + 16
    own_b = (kidx >= 0) & (kidx < HB2)
    kcl = jnp.clip(kidx, 0, HB2 - 1)
    plsc.store_scatter(sfl, [kcl], jnp.ones((L,), _i32), mask=own_b)

    @plsc.parallel_loop(0, C + L, L, unroll=2)
    def _red(k):
        q = ii + (k + L) + dd
        pv = ii + k + base
        ct = tbuf[pl.ds(jnp.minimum(k + L + dd, HB2 - L), L)]
        cf = sfl[pl.ds(k + L, L)]
        t1 = plsc.load_gather(tbuf, [jnp.clip(q - 1, 0, HB2 - 1)])
        t2 = plsc.load_gather(tbuf, [jnp.clip(q - 2, 0, HB2 - 1)])
        f1 = plsc.load_gather(sfl, [q - dd - 1])
        red = ((ct >= P) & (ct <= OP_MUL) & (t1 < P) & (t2 < P)
               & (cf == 0) & (f1 == 0) & (pv >= 2) & (pv < N))
        redf[pl.ds(k, L)] = red.astype(_i32)

    pf_ = jnp.float32(P)
    rcp = jnp.float32(1.0 / P)

    def _pass2(k, off):
        rq = ii + k
        q = rq + L
        r0 = redf[pl.ds(k, L)]
        t = tbuf[pl.ds(k + L + dd, L)]
        vf = vbuf[pl.ds(k + L + dd, L)]
        r1 = plsc.load_gather(redf, [rq + 1])
        r2 = plsc.load_gather(redf, [rq + 2])
        qd1 = jnp.clip(q + 1 + dd, 0, HB2 - 1)
        qd2 = jnp.clip(q + 2 + dd, 0, HB2 - 1)
        tn1 = plsc.load_gather(tbuf, [qd1])
        opc = plsc.load_gather(tbuf, [qd2])
        vn1 = plsc.load_gather(vbuf, [qd1])
        keep = (r0 == 0) & (r1 == 0)
        repl = r2 != 0
        radd = lax.rem(t + tn1, P)
        rsub = lax.rem(t - tn1 + P, P)
        rmul = lax.rem(t * tn1, P)
        res = jnp.where(opc == OP_ADD, radd,
                        jnp.where(opc == OP_SUB, rsub, rmul))
        tok_new = jnp.where(repl, res, t)
        fa = vf + vn1
        fa = fa - jnp.where(fa >= pf_, pf_, 0.0)
        fs = vf - vn1
        fs = fs + jnp.where(fs < 0.0, pf_, 0.0)
        fm0 = vf * vn1
        q = (fm0 * rcp).astype(_i32).astype(_f32)
        fm = fm0 - q * pf_
        fm = fm + jnp.where(fm < 0.0, pf_, 0.0)
        fm = fm - jnp.where(fm >= pf_, pf_, 0.0)
        resf = jnp.where(opc == OP_ADD, fa,
                         jnp.where(opc == OP_SUB, fs, fm))
        val_new = jnp.where(repl, resf, vf)
        k32 = keep.astype(_i32)
        incl = plsc.cumsum(k32)
        npexc = off + incl - k32
        npb[pl.ds(k, L)] = npexc
        plsc.store_scatter(ktok, [npexc], tok_new, mask=keep)
        plsc.store_scatter(kval, [npexc], val_new, mask=keep)
        return off + incl[15]
    cnt = plsc.parallel_loop(0, C, L, unroll=2, carry=jnp.int32(0))(_pass2)


    lidx = cuvec - base
    own = ((lidx >= 0) & (lidx < C)) | ((w == NT - 1) & (lidx == C))
    lcl = jnp.clip(lidx, 0, C - 1)
    gat = plsc.load_gather(npb, [lcl], mask=own)
    vwvec = jnp.where(lidx == C, cnt, jnp.where(own, gat, 0))

    f_t.wait()
    f_v.wait()

    plsc.subcore_barrier()
    plsc.fetch_and_add(smem.at[2 + w], cnt, subcore_id=0)
    plsc.subcore_barrier()

    @pl.when(w == 0)
    def _():
        counts = jnp.zeros((L,), _i32)
        for t in range(NT):
            counts = jnp.where(ii == t, smem[2 + t], counts)
        gbv0 = plsc.cumsum(counts) - counts
        for t in range(NT):
            gbt = jnp.sum(jnp.where(ii == t, gbv0, 0))
            plsc.fetch_and_add(smem.at[0], gbt, subcore_id=t)

    plsc.subcore_barrier()
    gb = smem[0]

    ncub[...] = jnp.where(own, vwvec + gb, 0)
    cuix[0, pl.ds(0, L)] = jnp.where(own, ii + 1, 0)
    handles = [pltpu.async_copy(ncub, ncu_hbm.at[cuix.at[0]], sem_sc)]

    a = lax.rem(8 - lax.rem(gb, 8), 8)
    gbu = pl.multiple_of(gb + a, 8)
    len8 = ((cnt - a) // 8) * 8
    rem = cnt - a - len8

    rank = jnp.where(ii < 8, ii, a + len8 + ii - 8)
    rankc = jnp.clip(rank, 0, C + 15)
    fragt[...] = plsc.load_gather(ktok, [rankc])
    fragv[...] = plsc.load_gather(kval, [rankc])
    validh = ii < jnp.minimum(a, 8)
    validt = (ii >= 8) & (ii - 8 < rem)
    fragix[0, pl.ds(0, L)] = jnp.where(
        validh, gb + ii, jnp.where(validt, gbu + len8 + ii - 8, N))
    handles.append(pltpu.async_copy(fragt, out_tok.at[fragix.at[0]], sem_sc))
    handles.append(pltpu.async_copy(fragv, out_val.at[fragix.at[0]], sem_sc))

    @plsc.parallel_loop(0, C, L, unroll=2)
    def _pass3(k):
        idxv = ii + k + a
        obt[pl.ds(k, L)] = plsc.load_gather(ktok, [idxv])
        obv[pl.ds(k, L)] = plsc.load_gather(kval, [idxv])

    s = jnp.int32(0)
    for b in (2048, 1024, 512, 256, 128, 64, 32, 16, 8):
        cond = (len8 & b) != 0
        dst = pl.multiple_of(jnp.where(cond, gbu + s, N + 16), 8)
        src = pl.multiple_of(jnp.where(cond, s, 0), 8)
        handles.append(pltpu.async_copy(
            obt.at[pl.ds(src, b)], out_tok.at[pl.ds(dst, b)], sem_sc))
        handles.append(pltpu.async_copy(
            obv.at[pl.ds(src, b)], out_val.at[pl.ds(dst, b)], sem_sc))
        s = s + (len8 & b)

    for h in handles:
        h.wait()


@jax.jit
def _rpn_sc(toks_p, vals_p, cu16):
    mesh = plsc.VectorSubcoreMesh(core_axis_name="c", subcore_axis_name="s",
                                  num_cores=1)
    fn = pl.kernel(
        _sc_body,
        mesh=mesh,
        compiler_params=pltpu.CompilerParams(needs_layout_passes=False),
        out_type=[
            jax.ShapeDtypeStruct((N + OPAD,), _i32),
            jax.ShapeDtypeStruct((N + OPAD,), _f32),
            jax.ShapeDtypeStruct((17,), _i32),
        ],
        scratch_types=[
            pltpu.VMEM((HB2,), _i32),
            pltpu.VMEM((HB2,), _f32),
            pltpu.VMEM((HB2,), _i32),
            pltpu.VMEM((C + 16,), _i32),
            pltpu.VMEM((C + 16,), _i32),
            pltpu.VMEM((C + 16,), _f32),
            pltpu.VMEM((C,), _i32),
            pltpu.VMEM((C + 16,), _i32),
            pltpu.VMEM((C + 16,), _f32),
            pltpu.VMEM((C,), _i32),
            pltpu.VMEM((C,), _f32),
            pltpu.VMEM((L,), _i32),
            pltpu.VMEM((1, L), _i32),
            pltpu.VMEM((L,), _i32),
            pltpu.VMEM((L,), _i32),
            pltpu.VMEM((L,), _f32),
            pltpu.VMEM((1, L), _i32),
            pltpu.SMEM((24,), _i32),
            pltpu.SemaphoreType.DMA,
            pltpu.SemaphoreType.DMA,
            pltpu.SemaphoreType.DMA,
        ],
    )
    return fn(toks_p, vals_p, cu16)


def kernel(tokens, cu_seqlens, values_f):
    cu16 = cu_seqlens[1:17]
    out_tok_p, out_val_p, new_cu = _rpn_sc(tokens, values_f, cu16)
    return out_tok_p[:N], out_val_p[:N], new_cu


# speedup vs baseline: 1.0187x; 1.0187x over previous
import functools
import jax
import jax.numpy as jnp
from jax import lax
from jax.experimental import pallas as pl
from jax.experimental.pallas import tpu as pltpu
from jax.experimental.pallas import tpu_sc as plsc

P = 97
OP_ADD = P
OP_SUB = P + 1
OP_MUL = P + 2
PAD = P + 6
N = 32768
NT = 16
C = N // NT
L = 16
NV = C // L
HB2 = C + 48
FPAD = 24
TPAD = 40
OPAD = 2080

_i32 = jnp.int32
_f32 = jnp.float32


def _sc_body(tok_hbm, val_hbm, cu_hbm, out_tok, out_val, ncu_hbm,
             tbuf, vbuf, sfl, redf, ktok, kval, npb, obt, obv,
             padb, zfb, cuv, cuix, ncub, fragt, fragv, fragix, smem,
             sem_in, sem_fill, sem_sc):
    w = lax.axis_index("s")
    base = pl.multiple_of(w * C, C)
    ii = lax.iota(_i32, L)
    zz = jnp.zeros((L,), _i32)

    def _zsm(i, _):
        smem[i] = 0
        return 0
    lax.fori_loop(0, 18, _zsm, 0)

    oin = pl.multiple_of(jnp.clip(w * C - 16, 0, N - HB2), 8)
    dd = (w * C - 16) - oin
    h_t = pltpu.async_copy(tok_hbm.at[pl.ds(oin, HB2)], tbuf, sem_in)
    h_v = pltpu.async_copy(val_hbm.at[pl.ds(oin, HB2)], vbuf, sem_in)
    pltpu.sync_copy(cu_hbm, cuv)

    @plsc.parallel_loop(0, C, L)
    def _prep(k):
        padb[pl.ds(k, L)] = jnp.full((L,), PAD, _i32)
        zfb[pl.ds(k, L)] = jnp.zeros((L,), _f32)
        sfl[pl.ds(k, L)] = zz
    @plsc.parallel_loop(C, HB2, L)
    def _prep2(k):
        sfl[pl.ds(k, L)] = zz

    h_t.wait()
    h_v.wait()

    f_t = pltpu.async_copy(padb, out_tok.at[pl.ds(base, C)], sem_fill)
    f_v = pltpu.async_copy(zfb, out_val.at[pl.ds(base, C)], sem_fill)

    cuvec = cuv[...]
    kidx = cuvec - base + 16
    own_b = (kidx >= 0) & (kidx < HB2)
    kcl = jnp.clip(kidx, 0, HB2 - 1)
    plsc.store_scatter(sfl, [kcl], jnp.ones((L,), _i32), mask=own_b)

    @plsc.parallel_loop(0, C + L, L)
    def _red(k):
        q = ii + (k + L) + dd
        pv = ii + k + base
        ct = tbuf[pl.ds(jnp.minimum(k + L + dd, HB2 - L), L)]
        cf = sfl[pl.ds(k + L, L)]
        t1 = plsc.load_gather(tbuf, [jnp.clip(q - 1, 0, HB2 - 1)])
        t2 = plsc.load_gather(tbuf, [jnp.clip(q - 2, 0, HB2 - 1)])
        f1 = plsc.load_gather(sfl, [q - dd - 1])
        red = ((ct >= P) & (ct <= OP_MUL) & (t1 < P) & (t2 < P)
               & (cf == 0) & (f1 == 0) & (pv >= 2) & (pv < N))
        redf[pl.ds(k, L)] = red.astype(_i32)

    pf_ = jnp.float32(P)
    rcp = jnp.float32(1.0 / P)

    def _pass2(k, off):
        rq = ii + k
        q = rq + L
        r0 = redf[pl.ds(k, L)]
        t = tbuf[pl.ds(k + L + dd, L)]
        vf = vbuf[pl.ds(k + L + dd, L)]
        r1 = plsc.load_gather(redf, [rq + 1])
        r2 = plsc.load_gather(redf, [rq + 2])
        qd1 = jnp.clip(q + 1 + dd, 0, HB2 - 1)
        qd2 = jnp.clip(q + 2 + dd, 0, HB2 - 1)
        tn1 = plsc.load_gather(tbuf, [qd1])
        opc = plsc.load_gather(tbuf, [qd2])
        vn1 = plsc.load_gather(vbuf, [qd1])
        keep = (r0 == 0) & (r1 == 0)
        repl = r2 != 0
        radd = lax.rem(t + tn1, P)
        rsub = lax.rem(t - tn1 + P, P)
        rmul = lax.rem(t * tn1, P)
        res = jnp.where(opc == OP_ADD, radd,
                        jnp.where(opc == OP_SUB, rsub, rmul))
        tok_new = jnp.where(repl, res, t)
        fa = vf + vn1
        fa = fa - jnp.where(fa >= pf_, pf_, 0.0)
        fs = vf - vn1
        fs = fs + jnp.where(fs < 0.0, pf_, 0.0)
        fm0 = vf * vn1
        q = (fm0 * rcp).astype(_i32).astype(_f32)
        fm = fm0 - q * pf_
        fm = fm + jnp.where(fm < 0.0, pf_, 0.0)
        fm = fm - jnp.where(fm >= pf_, pf_, 0.0)
        resf = jnp.where(opc == OP_ADD, fa,
                         jnp.where(opc == OP_SUB, fs, fm))
        val_new = jnp.where(repl, resf, vf)
        k32 = keep.astype(_i32)
        incl = plsc.cumsum(k32)
        npexc = off + incl - k32
        npb[pl.ds(k, L)] = npexc
        plsc.store_scatter(ktok, [npexc], tok_new, mask=keep)
        plsc.store_scatter(kval, [npexc], val_new, mask=keep)
        return off + incl[15]
    cnt = plsc.parallel_loop(0, C, L, carry=jnp.int32(0))(_pass2)


    lidx = cuvec - base
    own = ((lidx >= 0) & (lidx < C)) | ((w == NT - 1) & (lidx == C))
    lcl = jnp.clip(lidx, 0, C - 1)
    gat = plsc.load_gather(npb, [lcl], mask=own)
    vwvec = jnp.where(lidx == C, cnt, jnp.where(own, gat, 0))

    f_t.wait()
    f_v.wait()

    plsc.subcore_barrier()
    plsc.fetch_and_add(smem.at[2 + w], cnt, subcore_id=0)
    plsc.subcore_barrier()

    @pl.when(w == 0)
    def _():
        counts = jnp.zeros((L,), _i32)
        for t in range(NT):
            counts = jnp.where(ii == t, smem[2 + t], counts)
        gbv0 = plsc.cumsum(counts) - counts
        for t in range(NT):
            gbt = jnp.sum(jnp.where(ii == t, gbv0, 0))
            plsc.fetch_and_add(smem.at[0], gbt, subcore_id=t)

    plsc.subcore_barrier()
    gb = smem[0]

    ncub[...] = jnp.where(own, vwvec + gb, 0)
    cuix[0, pl.ds(0, L)] = jnp.where(own, ii + 1, 0)
    handles = [pltpu.async_copy(ncub, ncu_hbm.at[cuix.at[0]], sem_sc)]

    a = lax.rem(8 - lax.rem(gb, 8), 8)
    gbu = pl.multiple_of(gb + a, 8)
    len8 = ((cnt - a) // 8) * 8
    rem = cnt - a - len8

    rank = jnp.where(ii < 8, ii, a + len8 + ii - 8)
    rankc = jnp.clip(rank, 0, C + 15)
    fragt[...] = plsc.load_gather(ktok, [rankc])
    fragv[...] = plsc.load_gather(kval, [rankc])
    validh = ii < jnp.minimum(a, 8)
    validt = (ii >= 8) & (ii - 8 < rem)
    fragix[0, pl.ds(0, L)] = jnp.where(
        validh, gb + ii, jnp.where(validt, gbu + len8 + ii - 8, N))
    handles.append(pltpu.async_copy(fragt, out_tok.at[fragix.at[0]], sem_sc))
    handles.append(pltpu.async_copy(fragv, out_val.at[fragix.at[0]], sem_sc))

    @plsc.parallel_loop(0, C, L)
    def _pass3(k):
        idxv = ii + k + a
        obt[pl.ds(k, L)] = plsc.load_gather(ktok, [idxv])
        obv[pl.ds(k, L)] = plsc.load_gather(kval, [idxv])

    s = jnp.int32(0)
    for b in (2048, 1024, 512, 256, 128, 64, 32, 16, 8):
        cond = (len8 & b) != 0
        dst = pl.multiple_of(jnp.where(cond, gbu + s, N + 16), 8)
        src = pl.multiple_of(jnp.where(cond, s, 0), 8)
        handles.append(pltpu.async_copy(
            obt.at[pl.ds(src, b)], out_tok.at[pl.ds(dst, b)], sem_sc))
        handles.append(pltpu.async_copy(
            obv.at[pl.ds(src, b)], out_val.at[pl.ds(dst, b)], sem_sc))
        s = s + (len8 & b)

    for h in handles:
        h.wait()


@jax.jit
def _rpn_sc(toks_p, vals_p, cu16):
    mesh = plsc.VectorSubcoreMesh(core_axis_name="c", subcore_axis_name="s",
                                  num_cores=1)
    fn = pl.kernel(
        _sc_body,
        mesh=mesh,
        compiler_params=pltpu.CompilerParams(needs_layout_passes=False),
        out_type=[
            jax.ShapeDtypeStruct((N + OPAD,), _i32),
            jax.ShapeDtypeStruct((N + OPAD,), _f32),
            jax.ShapeDtypeStruct((17,), _i32),
        ],
        scratch_types=[
            pltpu.VMEM((HB2,), _i32),
            pltpu.VMEM((HB2,), _f32),
            pltpu.VMEM((HB2,), _i32),
            pltpu.VMEM((C + 16,), _i32),
            pltpu.VMEM((C + 16,), _i32),
            pltpu.VMEM((C + 16,), _f32),
            pltpu.VMEM((C,), _i32),
            pltpu.VMEM((C + 16,), _i32),
            pltpu.VMEM((C + 16,), _f32),
            pltpu.VMEM((C,), _i32),
            pltpu.VMEM((C,), _f32),
            pltpu.VMEM((L,), _i32),
            pltpu.VMEM((1, L), _i32),
            pltpu.VMEM((L,), _i32),
            pltpu.VMEM((L,), _i32),
            pltpu.VMEM((L,), _f32),
            pltpu.VMEM((1, L), _i32),
            pltpu.SMEM((24,), _i32),
            pltpu.SemaphoreType.DMA,
            pltpu.SemaphoreType.DMA,
            pltpu.SemaphoreType.DMA,
        ],
    )
    return fn(toks_p, vals_p, cu16)


def kernel(tokens, cu_seqlens, values_f):
    cu16 = cu_seqlens[1:17]
    out_tok_p, out_val_p, new_cu = _rpn_sc(tokens, values_f, cu16)
    return out_tok_p[:N], out_val_p[:N], new_cu
